# Initial kernel scaffold; baseline (speedup 1.0000x reference)
import jax
import jax.numpy as jnp
from jax.experimental import pallas as pl


def kernel(x, params, edge_index, batch):
    raise NotImplementedError("write your pallas kernel here")


# speedup vs baseline: 6.3720x; 6.3720x over previous
import functools

import jax
import jax.numpy as jnp
import numpy as np
from jax.experimental import pallas as pl
from jax.experimental.pallas import tpu as pltpu
from jax.experimental.pallas import tpu_sc as plsc

_K_LIST = [20, 11, 7, 6, 5]
_NB = 256
_BIG = 1e30
_BIG2 = 2e30
_BN_S = float(1.0 / np.sqrt(1.0 + 1e-5))


def _knn_body(k, n, xb_ref, xt_ref, bcol_ref, brow_ref, idx_ref):
    xb = xb_ref[...]
    xt = xt_ref[...]
    sq_col = jnp.sum(xb * xb, axis=1, keepdims=True)
    sq_row = jnp.sum(xt * xt, axis=0, keepdims=True)
    prod = jnp.dot(xb, xt, preferred_element_type=jnp.float32)
    d2 = sq_col + sq_row - 2.0 * prod
    mask = bcol_ref[...] != brow_ref[...]
    d2 = jnp.where(mask, jnp.float32(_BIG), d2)
    col = jax.lax.broadcasted_iota(jnp.int32, d2.shape, 1)
    for t in range(k):
        v = jnp.min(d2, axis=1, keepdims=True)
        cand = jnp.where(d2 <= v, col, jnp.int32(n))
        sel = jnp.min(cand, axis=1, keepdims=True)
        idx_ref[:, t : t + 1] = sel
        d2 = jnp.where(col == sel, jnp.float32(_BIG2), d2)


def _knn(x, xt, bcol, brow, k):
    n, d = x.shape
    nb = _NB
    return pl.pallas_call(
        functools.partial(_knn_body, k, n),
        grid=(n // nb,),
        in_specs=[
            pl.BlockSpec((nb, d), lambda i: (i, 0)),
            pl.BlockSpec((d, n), lambda i: (0, 0)),
            pl.BlockSpec((nb, 1), lambda i: (i, 0)),
            pl.BlockSpec((1, n), lambda i: (0, 0)),
        ],
        out_specs=pl.BlockSpec((nb, k), lambda i: (i, 0)),
        out_shape=jax.ShapeDtypeStruct((n, k), jnp.int32),
    )(x, xt, bcol, brow)


def _ab_body(xb_ref, wa_ref, b0_ref, wb_ref, a_ref, b_ref):
    xb = xb_ref[...]
    a_ref[...] = jnp.dot(xb, wa_ref[...], preferred_element_type=jnp.float32) + b0_ref[...]
    b_ref[...] = jnp.dot(xb, wb_ref[...], preferred_element_type=jnp.float32)


def _ab(x, wa, b0, wb):
    n, d = x.shape
    h = wa.shape[1]
    nb = _NB
    return pl.pallas_call(
        _ab_body,
        grid=(n // nb,),
        in_specs=[
            pl.BlockSpec((nb, d), lambda i: (i, 0)),
            pl.BlockSpec((d, h), lambda i: (0, 0)),
            pl.BlockSpec((1, h), lambda i: (0, 0)),
            pl.BlockSpec((d, h), lambda i: (0, 0)),
        ],
        out_specs=[
            pl.BlockSpec((nb, h), lambda i: (i, 0)),
            pl.BlockSpec((nb, h), lambda i: (i, 0)),
        ],
        out_shape=[
            jax.ShapeDtypeStruct((n, h), jnp.float32),
            jax.ShapeDtypeStruct((n, h), jnp.float32),
        ],
    )(x, wa, b0, wb)


def _linear_body(xb_ref, w_ref, b_ref, o_ref):
    o_ref[...] = (
        jnp.dot(xb_ref[...], w_ref[...], preferred_element_type=jnp.float32) + b_ref[...]
    )


def _linear(x, w, b):
    n, d = x.shape
    h = w.shape[1]
    nb = _NB
    return pl.pallas_call(
        _linear_body,
        grid=(n // nb,),
        in_specs=[
            pl.BlockSpec((nb, d), lambda i: (i, 0)),
            pl.BlockSpec((d, h), lambda i: (0, 0)),
            pl.BlockSpec((1, h), lambda i: (0, 0)),
        ],
        out_specs=pl.BlockSpec((nb, h), lambda i: (i, 0)),
        out_shape=jax.ShapeDtypeStruct((n, h), jnp.float32),
    )(x, w, b)


def _sc_gather(data, idx_flat):
    h = data.shape[1]
    kn = idx_flat.shape[1]
    win = 128
    mesh = plsc.VectorSubcoreMesh(core_axis_name="c", subcore_axis_name="s")

    @pl.kernel(out_type=jax.ShapeDtypeStruct((kn, h), data.dtype), mesh=mesh)
    def gather_kernel(d_hbm, i_hbm, o_hbm):
        def body(i_vmem, o_vmem):
            pltpu.sync_copy(d_hbm.at[i_vmem.at[0]], o_vmem)

        pltpu.emit_pipeline(
            body,
            grid=(kn // win,),
            in_specs=[pl.BlockSpec((1, win), lambda i: (0, i))],
            out_specs=[pl.BlockSpec((win, h), lambda i: (i, 0))],
            core_axis_name=("c", "s"),
            dimension_semantics=(pltpu.PARALLEL,),
        )(i_hbm, o_hbm)

    return gather_kernel(data, idx_flat)


def _edge_body(k, nblocks, has_skip, *refs):
    if has_skip:
        (g_ref, a_ref, w1_ref, b1_ref, w2_ref, b2_ref, sg_ref, sb_ref, skip_ref,
         out_ref) = refs
    else:
        g_ref, a_ref, w1_ref, b1_ref, w2_ref, b2_ref, sg_ref, sb_ref, out_ref = refs
    j = pl.program_id(1)
    h0 = jnp.maximum(g_ref[...] + a_ref[...], 0.0)
    h1 = jnp.maximum(
        jnp.dot(h0, w1_ref[...], preferred_element_type=jnp.float32) + b1_ref[...], 0.0
    )
    h2 = jnp.dot(h1, w2_ref[...], preferred_element_type=jnp.float32) + b2_ref[...]

    @pl.when(j == 0)
    def _():
        out_ref[...] = h2

    @pl.when(j > 0)
    def _():
        out_ref[...] = jnp.maximum(out_ref[...], h2)

    @pl.when(j == k - 1)
    def _():
        res = jnp.maximum(out_ref[...], 0.0) * sg_ref[...] + sb_ref[...]
        if has_skip:
            res = res + skip_ref[...]
        out_ref[...] = res


def _edge(g, a, w1, b1, w2, b2, sg, sb, skip, k):
    n, hp = a.shape
    h = w1.shape[1]
    nb = _NB
    nblocks = n // nb
    in_specs = [
        pl.BlockSpec((nb, hp), lambda i, j: (j * nblocks + i, 0)),
        pl.BlockSpec((nb, hp), lambda i, j: (i, 0)),
        pl.BlockSpec((hp, h), lambda i, j: (0, 0)),
        pl.BlockSpec((1, h), lambda i, j: (0, 0)),
        pl.BlockSpec((h, h), lambda i, j: (0, 0)),
        pl.BlockSpec((1, h), lambda i, j: (0, 0)),
        pl.BlockSpec((1, h), lambda i, j: (0, 0)),
        pl.BlockSpec((1, h), lambda i, j: (0, 0)),
    ]
    args = [g, a, w1, b1, w2, b2, sg, sb]
    if skip is not None:
        in_specs.append(pl.BlockSpec((nb, h), lambda i, j: (i, 0)))
        args.append(skip)
    return pl.pallas_call(
        functools.partial(_edge_body, k, nblocks, skip is not None),
        grid=(nblocks, k),
        in_specs=in_specs,
        out_specs=pl.BlockSpec((nb, h), lambda i, j: (i, 0)),
        out_shape=jax.ShapeDtypeStruct((n, h), jnp.float32),
    )(*args)


def _ln(v, g, b):
    m = jnp.mean(v, axis=1, keepdims=True)
    var = jnp.mean((v - m) ** 2, axis=1, keepdims=True)
    return (v - m) / jnp.sqrt(var + 1e-5) * g + b


def _pool_head_body(num_graphs, x_ref, bcol_ref, brow_ref,
                    w0_ref, b0_ref, ln0g_ref, ln0b_ref,
                    w1_ref, b1_ref, ln1g_ref, ln1b_ref,
                    w2_ref, b2_ref, ln2g_ref, ln2b_ref,
                    sk1w_ref, sk1b_ref, sk2w_ref, sk2b_ref,
                    ow_ref, ob_ref, out_ref, emb_ref):
    x = x_ref[...]
    bc = bcol_ref[...]
    br = brow_ref[...]
    g_col = jax.lax.broadcasted_iota(jnp.int32, (num_graphs, 1), 0).astype(jnp.float32)
    oh_t = (br == g_col).astype(jnp.float32)
    g_row = jax.lax.broadcasted_iota(jnp.int32, (1, num_graphs), 1).astype(jnp.float32)
    oh = (bc == g_row).astype(jnp.float32)
    cnt = jnp.dot(oh_t, jnp.ones_like(bc), preferred_element_type=jnp.float32)
    sums = jnp.dot(oh_t, x, preferred_element_type=jnp.float32)
    mean = sums / cnt
    mean_per_node = jnp.dot(oh, mean, preferred_element_type=jnp.float32)
    diff = x - mean_per_node
    ss = jnp.dot(oh_t, diff * diff, preferred_element_type=jnp.float32)
    std = jnp.sqrt(ss / jnp.maximum(cnt - 1.0, 1.0))
    maxes = []
    for gi in range(num_graphs):
        mg = jnp.max(
            jnp.where(bc == jnp.float32(gi), x, jnp.float32(-_BIG)),
            axis=0,
            keepdims=True,
        )
        maxes.append(mg)
    xmax = jnp.concatenate(maxes, axis=0)
    emb = jnp.concatenate([mean, xmax, std], axis=1)
    emb_ref[...] = emb

    h0 = jnp.dot(emb, w0_ref[...], preferred_element_type=jnp.float32) + b0_ref[...]
    h0 = jnp.maximum(_ln(h0, ln0g_ref[...], ln0b_ref[...]), 0.0)
    h1 = jnp.dot(h0, w1_ref[...], preferred_element_type=jnp.float32) + b1_ref[...]
    h1 = jnp.maximum(_ln(h1, ln1g_ref[...], ln1b_ref[...]), 0.0)
    h1 = h1 + jnp.dot(emb, sk1w_ref[...], preferred_element_type=jnp.float32) + sk1b_ref[...]
    h2 = jnp.dot(h1, w2_ref[...], preferred_element_type=jnp.float32) + b2_ref[...]
    h2 = jnp.maximum(_ln(h2, ln2g_ref[...], ln2b_ref[...]), 0.0)
    h2 = h2 + jnp.dot(emb, sk2w_ref[...], preferred_element_type=jnp.float32) + sk2b_ref[...]
    out_ref[...] = jnp.dot(h2, ow_ref[...], preferred_element_type=jnp.float32) + ob_ref[...]


def _pool_head(x, bcol, brow, hp, num_graphs):
    n, h = x.shape
    row = lambda v: v.reshape(1, -1)
    args = (
        x, bcol, brow,
        hp["l0"]["W"], row(hp["l0"]["b"]), row(hp["ln0_g"]), row(hp["ln0_b"]),
        hp["l1"]["W"], row(hp["l1"]["b"]), row(hp["ln1_g"]), row(hp["ln1_b"]),
        hp["l2"]["W"], row(hp["l2"]["b"]), row(hp["ln2_g"]), row(hp["ln2_b"]),
        hp["sk1"]["W"], row(hp["sk1"]["b"]), hp["sk2"]["W"], row(hp["sk2"]["b"]),
        hp["out"]["W"], row(hp["out"]["b"]),
    )
    return pl.pallas_call(
        functools.partial(_pool_head_body, num_graphs),
        out_shape=[
            jax.ShapeDtypeStruct((num_graphs, 1), jnp.float32),
            jax.ShapeDtypeStruct((num_graphs, 3 * h), jnp.float32),
        ],
    )(*args)


def kernel(x, params, edge_index, batch):
    del edge_index
    n = x.shape[0]
    num_graphs = 4
    bcol = batch.astype(jnp.float32).reshape(n, 1)
    brow = batch.astype(jnp.float32).reshape(1, n)
    s = jnp.float32(_BN_S)
    xc = x
    for i in range(5):
        p = params["convs"][i]
        k = _K_LIST[i]
        d = xc.shape[1]
        h = p["l0"]["W"].shape[1]
        hp = ((h + 127) // 128) * 128
        w0 = p["l0"]["W"]
        pad = ((0, 0), (0, hp - h))
        wa = jnp.pad(w0[:d] - w0[d:], pad)
        wb = jnp.pad(w0[d:], pad)
        b0 = jnp.pad(p["l0"]["b"], (0, hp - h)).reshape(1, hp)
        w1 = jnp.pad((s * p["bn_g"])[:, None] * p["l1"]["W"], ((0, hp - h), (0, 0)))
        b1 = (p["bn_b"] @ p["l1"]["W"] + p["l1"]["b"]).reshape(1, h)
        w2 = p["l2"]["W"]
        b2 = p["l2"]["b"].reshape(1, h)
        sg = (s * params["bns"][i]["g"]).reshape(1, h)
        sb = params["bns"][i]["b"].reshape(1, h)
        idx = _knn(xc, xc.T, bcol, brow, k)
        a, b = _ab(xc, wa, b0, wb)
        g = _sc_gather(b, idx.T.reshape(1, k * n))
        skip = None
        if i == 4:
            skip = _linear(xc, params["skip3"]["W"], params["skip3"]["b"].reshape(1, -1))
        xc = _edge(g, a, w1, b1, w2, b2, sg, sb, skip, k)
    out, emb = _pool_head(xc, bcol, brow, params["head"], num_graphs)
    return out, emb
